# baseline (speedup 1.0000x reference)
import dataclasses
import functools

import jax
import jax.numpy as jnp
from jax import lax
from jax.experimental import pallas as pl
from jax.experimental.pallas import tpu as pltpu
from jax.experimental.pallas import tpu_sc as plsc

N = 10000
E = 320000
F = 128

_MESH = plsc.VectorSubcoreMesh(
    core_axis_name="c", subcore_axis_name="s", num_cores=2, num_subcores=16
)

_SC_PARAMS = pltpu.CompilerParams()
if "needs_layout_passes" in pltpu.CompilerParams.__dataclass_fields__:
    _SC_PARAMS = dataclasses.replace(_SC_PARAMS, needs_layout_passes=False)


_EPT = E // 32
_DCH = 2000


def _deg_body(src_hbm, ew_hbm, degp_hbm, degp_v, sbuf, wbuf):
    cid = lax.axis_index("c")
    sid = lax.axis_index("s")
    wid = cid * 16 + sid

    @pl.loop(0, N, step=16)
    def _zero(i):
        degp_v[pl.ds(i, 16)] = jnp.zeros((16,), jnp.float32)

    base = wid * _EPT

    @pl.loop(0, _EPT, step=_DCH)
    def _chunk(off):
        pltpu.sync_copy(src_hbm.at[pl.ds(base + off, _DCH)], sbuf)
        pltpu.sync_copy(ew_hbm.at[pl.ds(base + off, _DCH)], wbuf)

        @pl.loop(0, _DCH, step=16)
        def _vec(k):
            idx16 = sbuf[pl.ds(k, 16)]
            w16 = wbuf[pl.ds(k, 16)]
            plsc.addupdate_scatter(degp_v, [idx16], w16)

    pltpu.sync_copy(degp_v, degp_hbm.at[pl.ds(wid * N, N)])


def _deg_partials(src, ew):
    kfn = pl.kernel(
        _deg_body,
        out_type=jax.ShapeDtypeStruct((32 * N,), jnp.float32),
        mesh=_MESH,
        scratch_types=[
            pltpu.VMEM((N,), jnp.float32),
            pltpu.VMEM((_DCH,), jnp.int32),
            pltpu.VMEM((_DCH,), jnp.float32),
        ],
        compiler_params=_SC_PARAMS,
    )
    return kfn(src, ew)


_DB = 400


def _dis_body(degp_ref, dis_ref):
    deg = jnp.sum(degp_ref[...], axis=0)
    dis_ref[...] = jnp.where(deg > 0, lax.rsqrt(deg), 0.0)


def _compute_dis(degp):
    return pl.pallas_call(
        _dis_body,
        out_shape=jax.ShapeDtypeStruct((N,), jnp.float32),
    )(degp)


def _prescale_body(dis_ref, x_ref, h_ref, xs_ref, hs_ref):
    dis = dis_ref[...]
    xs_ref[...] = dis * x_ref[...]
    hs_ref[...] = dis * h_ref[...]


def _prescale(dis_col, x, h):
    blk = pl.BlockSpec((_DB, F), lambda i: (i, 0))
    return pl.pallas_call(
        _prescale_body,
        grid=(N // _DB,),
        in_specs=[pl.BlockSpec((_DB, 1), lambda i: (i, 0)), blk, blk],
        out_specs=[blk, blk],
        out_shape=[
            jax.ShapeDtypeStruct((N, F), jnp.float32),
            jax.ShapeDtypeStruct((N, F), jnp.float32),
        ],
    )(dis_col, x, h)


_CH = 80
_NROW = E // _CH
_SCR = 32
_NSC = _NROW // _SCR
_NBUF = 4
_ZCH = _CH
_NZC = N // _ZCH


def _scat_body(x_hbm, h_hbm, src_hbm, dst_hbm, ew_hbm,
               ox_hbm, oh_hbm,
               acc_sh, sbuf, dbuf, wbuf,
               rows, isem, gsems, ssems):
    cid = lax.axis_index("c")
    sid = lax.axis_index("s")

    @pl.loop(0, _ZCH)
    def _zrow(r):
        for k in range(F // 16):
            rows[0][r, pl.ds(k * 16, 16)] = jnp.zeros((16,), jnp.float32)

    @pl.loop(sid, _NZC, step=16)
    def _zcp(b):
        pltpu.sync_copy(rows[0], acc_sh.at[pl.ds(b * _ZCH, _ZCH)])

    plsc.subcore_barrier()

    def run(xin_hbm, out_hbm):
        @pl.loop(sid, _NSC, step=16)
        def _super(c):
            r0 = c * _SCR
            c1 = pltpu.async_copy(
                src_hbm.at[pl.ds(r0 * _CH, _SCR * _CH)], sbuf, isem)
            c2 = pltpu.async_copy(dst_hbm.at[pl.ds(r0, _SCR)], dbuf, isem)
            c3 = pltpu.async_copy(
                ew_hbm.at[pl.ds(r0 * _CH, _SCR * _CH)], wbuf, isem)
            c1.wait(); c2.wait(); c3.wait()

            for b in range(_NBUF):
                pltpu.async_copy(
                    xin_hbm.at[sbuf.at[pl.ds(b * _CH, _CH)]], rows[b],
                    gsems[b])

            @pl.loop(0, _SCR // _NBUF)
            def _round(r):
                for b in range(_NBUF):
                    j = r * _NBUF + b
                    pltpu.make_async_copy(
                        xin_hbm.at[sbuf.at[pl.ds(j * _CH, _CH)]], rows[b],
                        gsems[b]).wait()

                    @pl.loop(0, _CH, unroll=2)
                    def _scale(jj):
                        jv = jnp.full((16,), jj, dtype=jnp.int32)
                        nj = plsc.load_gather(
                            wbuf.at[pl.ds(j * _CH, _CH)], [jv])
                        for k in range(F // 16):
                            sl = (jj, pl.ds(k * 16, 16))
                            rows[b][sl] = rows[b][sl] * nj

                    pltpu.async_copy(
                        rows[b], acc_sh.at[dbuf.at[j]], ssems[b], add=True)

                @pl.when(r < _SCR // _NBUF - 1)
                def _prefetch():
                    for b in range(_NBUF):
                        j = (r + 1) * _NBUF + b
                        pltpu.make_async_copy(
                            rows[b], acc_sh.at[dbuf.at[j - _NBUF]],
                            ssems[b]).wait()
                        pltpu.async_copy(
                            xin_hbm.at[sbuf.at[pl.ds(j * _CH, _CH)]],
                            rows[b], gsems[b])

            for b in range(_NBUF):
                j = _SCR - _NBUF + b
                pltpu.make_async_copy(
                    rows[b], acc_sh.at[dbuf.at[j]], ssems[b]).wait()

        plsc.subcore_barrier()

        @pl.loop(sid, _NZC, step=16)
        def _out(b):
            r0 = b * _ZCH
            pltpu.sync_copy(acc_sh.at[pl.ds(r0, _ZCH)],
                            out_hbm.at[pl.ds(r0, _ZCH)])

    @pl.when(cid == 0)
    def _():
        run(x_hbm, ox_hbm)

    @pl.when(cid == 1)
    def _():
        run(h_hbm, oh_hbm)


def _edge_pass(xs, hs, src2, dst2, ew2):
    kfn = pl.kernel(
        _scat_body,
        out_type=(
            jax.ShapeDtypeStruct((N, F), jnp.float32),
            jax.ShapeDtypeStruct((N, F), jnp.float32),
        ),
        mesh=_MESH,
        scratch_types=[
            pltpu.VMEM_SHARED((N, F), jnp.float32),
            pltpu.VMEM((_SCR * _CH,), jnp.int32),
            pltpu.VMEM((_SCR, _CH), jnp.int32),
            pltpu.VMEM((_SCR * _CH,), jnp.float32),
            [pltpu.VMEM((_CH, F), jnp.float32) for _ in range(_NBUF)],
            pltpu.SemaphoreType.DMA,
            [pltpu.SemaphoreType.DMA for _ in range(_NBUF)],
            [pltpu.SemaphoreType.DMA for _ in range(_NBUF)],
        ],
        compiler_params=_SC_PARAMS,
    )
    return kfn(xs, hs, src2, dst2, ew2)


_RB = 400


def _dense_body(x_ref, tx_ref, h_ref, th_ref, c_ref, dis_ref,
                w0_ref, w1_ref, w2_ref, w3_ref, b_ref, wc_ref, wl_ref, bl_ref,
                out_ref, hn_ref, cn_ref):
    dot = functools.partial(
        jnp.dot,
        precision=lax.Precision.HIGHEST,
        preferred_element_type=jnp.float32,
    )
    ndis = -dis_ref[...]
    g = (dot(x_ref[...], w0_ref[...]) + dot(ndis * tx_ref[...], w1_ref[...])
         + dot(h_ref[...], w2_ref[...]) + dot(ndis * th_ref[...], w3_ref[...])
         + b_ref[...])
    c_old = c_ref[...]
    wc = wc_ref[...]
    gi = jax.nn.sigmoid(g[:, 0:F] + wc[0:1, :] * c_old)
    gf = jax.nn.sigmoid(g[:, F:2 * F] + wc[1:2, :] * c_old)
    gt = jnp.tanh(g[:, 2 * F:3 * F])
    c_new = gf * c_old + gi * gt
    go = jax.nn.sigmoid(g[:, 3 * F:4 * F] + wc[2:3, :] * c_new)
    h_new = go * jnp.tanh(c_new)
    cn_ref[...] = c_new
    hn_ref[...] = h_new
    out_ref[...] = dot(h_new, wl_ref[...]) + bl_ref[...]


def _dense(x, tx1x, h, tx1h, c, dis, w0, w1, w2, w3, bias, wc, wl, bl):
    nblk = N // _RB
    row_spec = pl.BlockSpec((_RB, F), lambda i: (i, 0))
    full = lambda shape: pl.BlockSpec(shape, lambda i: (0,) * len(shape))
    return pl.pallas_call(
        _dense_body,
        grid=(nblk,),
        in_specs=[
            row_spec, row_spec, row_spec, row_spec, row_spec,
            pl.BlockSpec((_RB, 1), lambda i: (i, 0)),
            full((F, 4 * F)), full((F, 4 * F)), full((F, 4 * F)), full((F, 4 * F)),
            full((1, 4 * F)), full((3, F)), full((F, 1)), full((1, 1)),
        ],
        out_specs=[
            pl.BlockSpec((_RB, 1), lambda i: (i, 0)),
            row_spec, row_spec,
        ],
        out_shape=[
            jax.ShapeDtypeStruct((N, 1), jnp.float32),
            jax.ShapeDtypeStruct((N, F), jnp.float32),
            jax.ShapeDtypeStruct((N, F), jnp.float32),
        ],
    )(x, tx1x, h, tx1h, c, dis, w0, w1, w2, w3, bias, wc, wl, bl)


def kernel(x, edge_index, edge_weight, h, c, Wx0, Wx1, bx, Wh0, Wh1, bh,
           wc, bg, W_lin, b_lin):
    src = edge_index[0]
    dst = edge_index[1]

    degp = _deg_partials(src, edge_weight).reshape(32, N)
    dis = _compute_dis(degp).reshape(N, 1)
    xs, hs = _prescale(dis, x, h)
    dst2 = dst.reshape(_NROW, _CH)
    tx1x, tx1h = _edge_pass(xs, hs, src, dst2, edge_weight)

    w0 = jnp.transpose(Wx0, (1, 0, 2)).reshape(F, 4 * F)
    w1 = jnp.transpose(Wx1, (1, 0, 2)).reshape(F, 4 * F)
    w2 = jnp.transpose(Wh0, (1, 0, 2)).reshape(F, 4 * F)
    w3 = jnp.transpose(Wh1, (1, 0, 2)).reshape(F, 4 * F)
    bias = (bx + bh + bg).reshape(1, 4 * F)
    bl = b_lin.reshape(1, 1)

    out, h_new, c_new = _dense(x, tx1x, h, tx1h, c, dis, w0, w1, w2, w3,
                               bias, wc, W_lin, bl)
    return (out, h_new, c_new)


# speedup vs baseline: 1.0629x; 1.0629x over previous
import dataclasses
import functools

import jax
import jax.numpy as jnp
from jax import lax
from jax.experimental import pallas as pl
from jax.experimental.pallas import tpu as pltpu
from jax.experimental.pallas import tpu_sc as plsc

N = 10000
E = 320000
F = 128

_MESH = plsc.VectorSubcoreMesh(
    core_axis_name="c", subcore_axis_name="s", num_cores=2, num_subcores=16
)

_SC_PARAMS = pltpu.CompilerParams()
if "needs_layout_passes" in pltpu.CompilerParams.__dataclass_fields__:
    _SC_PARAMS = dataclasses.replace(_SC_PARAMS, needs_layout_passes=False)


_EPT = E // 32
_DCH = 2000


def _deg_body(src_hbm, ew_hbm, degp_hbm, degp_v, sbuf, wbuf):
    cid = lax.axis_index("c")
    sid = lax.axis_index("s")
    wid = cid * 16 + sid

    @pl.loop(0, N, step=16)
    def _zero(i):
        degp_v[pl.ds(i, 16)] = jnp.zeros((16,), jnp.float32)

    base = wid * _EPT

    @pl.loop(0, _EPT, step=_DCH)
    def _chunk(off):
        pltpu.sync_copy(src_hbm.at[pl.ds(base + off, _DCH)], sbuf)
        pltpu.sync_copy(ew_hbm.at[pl.ds(base + off, _DCH)], wbuf)

        @pl.loop(0, _DCH, step=16)
        def _vec(k):
            idx16 = sbuf[pl.ds(k, 16)]
            w16 = wbuf[pl.ds(k, 16)]
            plsc.addupdate_scatter(degp_v, [idx16], w16)

    pltpu.sync_copy(degp_v, degp_hbm.at[pl.ds(wid * N, N)])


def _deg_partials(src, ew):
    kfn = pl.kernel(
        _deg_body,
        out_type=jax.ShapeDtypeStruct((32 * N,), jnp.float32),
        mesh=_MESH,
        scratch_types=[
            pltpu.VMEM((N,), jnp.float32),
            pltpu.VMEM((_DCH,), jnp.int32),
            pltpu.VMEM((_DCH,), jnp.float32),
        ],
        compiler_params=_SC_PARAMS,
    )
    return kfn(src, ew)


_DB = 400


def _dis_body(degp_ref, dis_ref):
    deg = jnp.sum(degp_ref[...], axis=0)
    dis_ref[...] = jnp.where(deg > 0, lax.rsqrt(deg), 0.0)


def _compute_dis(degp):
    return pl.pallas_call(
        _dis_body,
        out_shape=jax.ShapeDtypeStruct((N,), jnp.float32),
    )(degp)


def _prescale_body(dis_ref, x_ref, h_ref, xs_ref, hs_ref):
    dis = dis_ref[...]
    xs_ref[...] = dis * x_ref[...]
    hs_ref[...] = dis * h_ref[...]


def _prescale(dis_col, x, h):
    blk = pl.BlockSpec((_DB, F), lambda i: (i, 0))
    return pl.pallas_call(
        _prescale_body,
        grid=(N // _DB,),
        in_specs=[pl.BlockSpec((_DB, 1), lambda i: (i, 0)), blk, blk],
        out_specs=[blk, blk],
        out_shape=[
            jax.ShapeDtypeStruct((N, F), jnp.float32),
            jax.ShapeDtypeStruct((N, F), jnp.float32),
        ],
    )(dis_col, x, h)


_CH = 80
_NROW = E // _CH
_SCR = 32
_NSC = _NROW // _SCR
_NBUF = 4
_ZCH = _CH
_NZC = N // _ZCH


def _scat_body(x_hbm, h_hbm, src_hbm, dst_hbm, ew_hbm,
               ox_hbm, oh_hbm,
               acc_sh, sbuf, dbuf, wbuf,
               rows, isem, gsems, ssems):
    cid = lax.axis_index("c")
    sid = lax.axis_index("s")

    @pl.loop(0, _ZCH)
    def _zrow(r):
        for k in range(F // 16):
            rows[0][r, pl.ds(k * 16, 16)] = jnp.zeros((16,), jnp.float32)

    @pl.loop(sid, _NZC, step=16)
    def _zcp(b):
        pltpu.sync_copy(rows[0], acc_sh.at[pl.ds(b * _ZCH, _ZCH)])

    plsc.subcore_barrier()

    def run(xin_hbm, out_hbm):
        @pl.loop(sid, _NSC, step=16)
        def _super(c):
            r0 = c * _SCR
            c1 = pltpu.async_copy(
                src_hbm.at[pl.ds(r0 * _CH, _SCR * _CH)], sbuf, isem)
            c2 = pltpu.async_copy(dst_hbm.at[pl.ds(r0, _SCR)], dbuf, isem)
            c3 = pltpu.async_copy(
                ew_hbm.at[pl.ds(r0 * _CH, _SCR * _CH)], wbuf, isem)
            c1.wait(); c2.wait(); c3.wait()

            for b in range(_NBUF):
                pltpu.async_copy(
                    xin_hbm.at[sbuf.at[pl.ds(b * _CH, _CH)]], rows[b],
                    gsems[b])

            @pl.loop(0, _SCR // _NBUF)
            def _round(r):
                for b in range(_NBUF):
                    j = r * _NBUF + b
                    pltpu.make_async_copy(
                        xin_hbm.at[sbuf.at[pl.ds(j * _CH, _CH)]], rows[b],
                        gsems[b]).wait()

                    @pl.loop(0, _CH, unroll=2)
                    def _scale(jj):
                        jv = jnp.full((16,), jj, dtype=jnp.int32)
                        nj = plsc.load_gather(
                            wbuf.at[pl.ds(j * _CH, _CH)], [jv])
                        for k in range(F // 16):
                            sl = (jj, pl.ds(k * 16, 16))
                            rows[b][sl] = rows[b][sl] * nj

                    pltpu.async_copy(
                        rows[b], acc_sh.at[dbuf.at[j]], ssems[b], add=True)

                @pl.when(r < _SCR // _NBUF - 1)
                def _prefetch():
                    for b in range(_NBUF):
                        j = (r + 1) * _NBUF + b
                        pltpu.make_async_copy(
                            rows[b], acc_sh.at[dbuf.at[j - _NBUF]],
                            ssems[b]).wait()
                        pltpu.async_copy(
                            xin_hbm.at[sbuf.at[pl.ds(j * _CH, _CH)]],
                            rows[b], gsems[b])

            for b in range(_NBUF):
                j = _SCR - _NBUF + b
                pltpu.make_async_copy(
                    rows[b], acc_sh.at[dbuf.at[j]], ssems[b]).wait()

        plsc.subcore_barrier()

        @pl.loop(sid, _NZC, step=16)
        def _out(b):
            r0 = b * _ZCH
            pltpu.sync_copy(acc_sh.at[pl.ds(r0, _ZCH)],
                            out_hbm.at[pl.ds(r0, _ZCH)])

    @pl.when(cid == 0)
    def _():
        run(x_hbm, ox_hbm)

    @pl.when(cid == 1)
    def _():
        run(h_hbm, oh_hbm)


def _edge_pass(xs, hs, src2, dst2, ew2):
    kfn = pl.kernel(
        _scat_body,
        out_type=(
            jax.ShapeDtypeStruct((N, F), jnp.float32),
            jax.ShapeDtypeStruct((N, F), jnp.float32),
        ),
        mesh=_MESH,
        scratch_types=[
            pltpu.VMEM_SHARED((N, F), jnp.float32),
            pltpu.VMEM((_SCR * _CH,), jnp.int32),
            pltpu.VMEM((_SCR, _CH), jnp.int32),
            pltpu.VMEM((_SCR * _CH,), jnp.float32),
            [pltpu.VMEM((_CH, F), jnp.float32) for _ in range(_NBUF)],
            pltpu.SemaphoreType.DMA,
            [pltpu.SemaphoreType.DMA for _ in range(_NBUF)],
            [pltpu.SemaphoreType.DMA for _ in range(_NBUF)],
        ],
        compiler_params=_SC_PARAMS,
    )
    return kfn(xs, hs, src2, dst2, ew2)


_RB = 400


_DOT = functools.partial(
    jnp.dot,
    precision=lax.Precision.HIGHEST,
    preferred_element_type=jnp.float32,
)


def _dense0_body(x_ref, h_ref, w0_ref, w2_ref, b_ref, g0_ref):
    g0_ref[...] = (_DOT(x_ref[...], w0_ref[...])
                   + _DOT(h_ref[...], w2_ref[...]) + b_ref[...])


def _dense0(x, h, w0, w2, bias):
    row_spec = pl.BlockSpec((_RB, F), lambda i: (i, 0))
    full = lambda shape: pl.BlockSpec(shape, lambda i: (0,) * len(shape))
    return pl.pallas_call(
        _dense0_body,
        grid=(N // _RB,),
        in_specs=[row_spec, row_spec,
                  full((F, 4 * F)), full((F, 4 * F)), full((1, 4 * F))],
        out_specs=pl.BlockSpec((_RB, 4 * F), lambda i: (i, 0)),
        out_shape=jax.ShapeDtypeStruct((N, 4 * F), jnp.float32),
    )(x, h, w0, w2, bias)


def _dense_body(g0_ref, tx_ref, th_ref, c_ref, dis_ref,
                w1_ref, w3_ref, wc_ref, wl_ref, bl_ref,
                out_ref, hn_ref, cn_ref):
    dot = _DOT
    ndis = -dis_ref[...]
    g = (g0_ref[...] + dot(ndis * tx_ref[...], w1_ref[...])
         + dot(ndis * th_ref[...], w3_ref[...]))
    c_old = c_ref[...]
    wc = wc_ref[...]
    gi = jax.nn.sigmoid(g[:, 0:F] + wc[0:1, :] * c_old)
    gf = jax.nn.sigmoid(g[:, F:2 * F] + wc[1:2, :] * c_old)
    gt = jnp.tanh(g[:, 2 * F:3 * F])
    c_new = gf * c_old + gi * gt
    go = jax.nn.sigmoid(g[:, 3 * F:4 * F] + wc[2:3, :] * c_new)
    h_new = go * jnp.tanh(c_new)
    cn_ref[...] = c_new
    hn_ref[...] = h_new
    out_ref[...] = dot(h_new, wl_ref[...]) + bl_ref[...]


def _dense(g0, tx1x, tx1h, c, dis, w1, w3, wc, wl, bl):
    nblk = N // _RB
    row_spec = pl.BlockSpec((_RB, F), lambda i: (i, 0))
    full = lambda shape: pl.BlockSpec(shape, lambda i: (0,) * len(shape))
    return pl.pallas_call(
        _dense_body,
        grid=(nblk,),
        in_specs=[
            pl.BlockSpec((_RB, 4 * F), lambda i: (i, 0)),
            row_spec, row_spec, row_spec,
            pl.BlockSpec((_RB, 1), lambda i: (i, 0)),
            full((F, 4 * F)), full((F, 4 * F)),
            full((3, F)), full((F, 1)), full((1, 1)),
        ],
        out_specs=[
            pl.BlockSpec((_RB, 1), lambda i: (i, 0)),
            row_spec, row_spec,
        ],
        out_shape=[
            jax.ShapeDtypeStruct((N, 1), jnp.float32),
            jax.ShapeDtypeStruct((N, F), jnp.float32),
            jax.ShapeDtypeStruct((N, F), jnp.float32),
        ],
    )(g0, tx1x, tx1h, c, dis, w1, w3, wc, wl, bl)


def kernel(x, edge_index, edge_weight, h, c, Wx0, Wx1, bx, Wh0, Wh1, bh,
           wc, bg, W_lin, b_lin):
    src = edge_index[0]
    dst = edge_index[1]

    w0 = jnp.transpose(Wx0, (1, 0, 2)).reshape(F, 4 * F)
    w1 = jnp.transpose(Wx1, (1, 0, 2)).reshape(F, 4 * F)
    w2 = jnp.transpose(Wh0, (1, 0, 2)).reshape(F, 4 * F)
    w3 = jnp.transpose(Wh1, (1, 0, 2)).reshape(F, 4 * F)
    bias = (bx + bh + bg).reshape(1, 4 * F)
    bl = b_lin.reshape(1, 1)

    degp = _deg_partials(src, edge_weight).reshape(32, N)
    dis = _compute_dis(degp).reshape(N, 1)
    xs, hs = _prescale(dis, x, h)
    dst2 = dst.reshape(_NROW, _CH)
    tx1x, tx1h = _edge_pass(xs, hs, src, dst2, edge_weight)
    g0 = _dense0(x, h, w0, w2, bias)

    out, h_new, c_new = _dense(g0, tx1x, tx1h, c, dis, w1, w3, wc, W_lin, bl)
    return (out, h_new, c_new)


# speedup vs baseline: 1.1245x; 1.0579x over previous
import dataclasses
import functools

import jax
import jax.numpy as jnp
from jax import lax
from jax.experimental import pallas as pl
from jax.experimental.pallas import tpu as pltpu
from jax.experimental.pallas import tpu_sc as plsc

N = 10000
E = 320000
F = 128

_MESH = plsc.VectorSubcoreMesh(
    core_axis_name="c", subcore_axis_name="s", num_cores=2, num_subcores=16
)

_SC_PARAMS = pltpu.CompilerParams()
if "needs_layout_passes" in pltpu.CompilerParams.__dataclass_fields__:
    _SC_PARAMS = dataclasses.replace(_SC_PARAMS, needs_layout_passes=False)


_EPT = E // 32
_DCH = 2000


def _deg_body(src_hbm, ew_hbm, degp_hbm, degp_v, sbuf, wbuf):
    cid = lax.axis_index("c")
    sid = lax.axis_index("s")
    wid = cid * 16 + sid

    @pl.loop(0, N, step=16)
    def _zero(i):
        degp_v[pl.ds(i, 16)] = jnp.zeros((16,), jnp.float32)

    base = wid * _EPT

    @pl.loop(0, _EPT, step=_DCH)
    def _chunk(off):
        pltpu.sync_copy(src_hbm.at[pl.ds(base + off, _DCH)], sbuf)
        pltpu.sync_copy(ew_hbm.at[pl.ds(base + off, _DCH)], wbuf)

        @pl.loop(0, _DCH, step=16)
        def _vec(k):
            idx16 = sbuf[pl.ds(k, 16)]
            w16 = wbuf[pl.ds(k, 16)]
            plsc.addupdate_scatter(degp_v, [idx16], w16)

    pltpu.sync_copy(degp_v, degp_hbm.at[pl.ds(wid * N, N)])


def _deg_partials(src, ew):
    kfn = pl.kernel(
        _deg_body,
        out_type=jax.ShapeDtypeStruct((32 * N,), jnp.float32),
        mesh=_MESH,
        scratch_types=[
            pltpu.VMEM((N,), jnp.float32),
            pltpu.VMEM((_DCH,), jnp.int32),
            pltpu.VMEM((_DCH,), jnp.float32),
        ],
        compiler_params=_SC_PARAMS,
    )
    return kfn(src, ew)


_DB = 400


def _dis_body(degp_ref, dis_ref):
    deg = jnp.sum(degp_ref[...], axis=0)
    dis_ref[...] = jnp.where(deg > 0, lax.rsqrt(deg), 0.0)


def _compute_dis(degp):
    return pl.pallas_call(
        _dis_body,
        out_shape=jax.ShapeDtypeStruct((N,), jnp.float32),
    )(degp)


def _prescale_body(dis_ref, x_ref, h_ref, xs_ref, hs_ref):
    dis = dis_ref[...]
    xs_ref[...] = dis * x_ref[...]
    hs_ref[...] = dis * h_ref[...]


def _prescale(dis_col, x, h):
    blk = pl.BlockSpec((_DB, F), lambda i: (i, 0))
    return pl.pallas_call(
        _prescale_body,
        grid=(N // _DB,),
        in_specs=[pl.BlockSpec((_DB, 1), lambda i: (i, 0)), blk, blk],
        out_specs=[blk, blk],
        out_shape=[
            jax.ShapeDtypeStruct((N, F), jnp.float32),
            jax.ShapeDtypeStruct((N, F), jnp.float32),
        ],
    )(dis_col, x, h)


_CH = 80
_NROW = E // _CH
_SCR = 32
_NSC = _NROW // _SCR
_NBUF = 4
_ZCH = _CH
_NZC = N // _ZCH


def _scat_body(x_hbm, h_hbm, src_hbm, dst_hbm, ew_hbm,
               ox_hbm, oh_hbm,
               acc_sh, sbuf, dbuf, wbuf,
               rows, isem, gsems, ssems):
    cid = lax.axis_index("c")
    sid = lax.axis_index("s")

    @pl.loop(0, _ZCH)
    def _zrow(r):
        for k in range(F // 16):
            rows[0][r, pl.ds(k * 16, 16)] = jnp.zeros((16,), jnp.float32)

    @pl.loop(sid, _NZC, step=16)
    def _zcp(b):
        pltpu.sync_copy(rows[0], acc_sh.at[pl.ds(b * _ZCH, _ZCH)])

    plsc.subcore_barrier()

    def run(xin_hbm, out_hbm):
        @pl.loop(sid, _NSC, step=16)
        def _super(c):
            r0 = c * _SCR
            c1 = pltpu.async_copy(
                src_hbm.at[pl.ds(r0 * _CH, _SCR * _CH)], sbuf, isem)
            c2 = pltpu.async_copy(dst_hbm.at[pl.ds(r0, _SCR)], dbuf, isem)
            c3 = pltpu.async_copy(
                ew_hbm.at[pl.ds(r0 * _CH, _SCR * _CH)], wbuf, isem)
            c1.wait(); c2.wait(); c3.wait()

            for b in range(_NBUF):
                pltpu.async_copy(
                    xin_hbm.at[sbuf.at[pl.ds(b * _CH, _CH)]], rows[b],
                    gsems[b])

            @pl.loop(0, _SCR // _NBUF)
            def _round(r):
                for b in range(_NBUF):
                    j = r * _NBUF + b
                    pltpu.make_async_copy(
                        xin_hbm.at[sbuf.at[pl.ds(j * _CH, _CH)]], rows[b],
                        gsems[b]).wait()

                    @pl.loop(0, _CH, unroll=4)
                    def _scale(jj):
                        jv = jnp.full((16,), jj, dtype=jnp.int32)
                        nj = plsc.load_gather(
                            wbuf.at[pl.ds(j * _CH, _CH)], [jv])
                        for k in range(F // 16):
                            sl = (jj, pl.ds(k * 16, 16))
                            rows[b][sl] = rows[b][sl] * nj

                    pltpu.async_copy(
                        rows[b], acc_sh.at[dbuf.at[j]], ssems[b], add=True)

                @pl.when(r < _SCR // _NBUF - 1)
                def _prefetch():
                    for b in range(_NBUF):
                        j = (r + 1) * _NBUF + b
                        pltpu.make_async_copy(
                            rows[b], acc_sh.at[dbuf.at[j - _NBUF]],
                            ssems[b]).wait()
                        pltpu.async_copy(
                            xin_hbm.at[sbuf.at[pl.ds(j * _CH, _CH)]],
                            rows[b], gsems[b])

            for b in range(_NBUF):
                j = _SCR - _NBUF + b
                pltpu.make_async_copy(
                    rows[b], acc_sh.at[dbuf.at[j]], ssems[b]).wait()

        plsc.subcore_barrier()

        @pl.loop(sid, _NZC, step=16)
        def _out(b):
            r0 = b * _ZCH
            pltpu.sync_copy(acc_sh.at[pl.ds(r0, _ZCH)],
                            out_hbm.at[pl.ds(r0, _ZCH)])

    @pl.when(cid == 0)
    def _():
        run(x_hbm, ox_hbm)

    @pl.when(cid == 1)
    def _():
        run(h_hbm, oh_hbm)


def _edge_pass(xs, hs, src2, dst2, ew2):
    kfn = pl.kernel(
        _scat_body,
        out_type=(
            jax.ShapeDtypeStruct((N, F), jnp.float32),
            jax.ShapeDtypeStruct((N, F), jnp.float32),
        ),
        mesh=_MESH,
        scratch_types=[
            pltpu.VMEM_SHARED((N, F), jnp.float32),
            pltpu.VMEM((_SCR * _CH,), jnp.int32),
            pltpu.VMEM((_SCR, _CH), jnp.int32),
            pltpu.VMEM((_SCR * _CH,), jnp.float32),
            [pltpu.VMEM((_CH, F), jnp.float32) for _ in range(_NBUF)],
            pltpu.SemaphoreType.DMA,
            [pltpu.SemaphoreType.DMA for _ in range(_NBUF)],
            [pltpu.SemaphoreType.DMA for _ in range(_NBUF)],
        ],
        compiler_params=_SC_PARAMS,
    )
    return kfn(xs, hs, src2, dst2, ew2)


_RB = 400


_DOT = functools.partial(
    jnp.dot,
    precision=lax.Precision.DEFAULT,
    preferred_element_type=jnp.float32,
)


def _dense0_body(x_ref, h_ref, w0_ref, w2_ref, b_ref, g0_ref):
    g0_ref[...] = (_DOT(x_ref[...], w0_ref[...])
                   + _DOT(h_ref[...], w2_ref[...]) + b_ref[...])


def _dense0(x, h, w0, w2, bias):
    row_spec = pl.BlockSpec((_RB, F), lambda i: (i, 0))
    full = lambda shape: pl.BlockSpec(shape, lambda i: (0,) * len(shape))
    return pl.pallas_call(
        _dense0_body,
        grid=(N // _RB,),
        in_specs=[row_spec, row_spec,
                  full((F, 4 * F)), full((F, 4 * F)), full((1, 4 * F))],
        out_specs=pl.BlockSpec((_RB, 4 * F), lambda i: (i, 0)),
        out_shape=jax.ShapeDtypeStruct((N, 4 * F), jnp.float32),
    )(x, h, w0, w2, bias)


def _dense_body(g0_ref, tx_ref, th_ref, c_ref, dis_ref,
                w1_ref, w3_ref, wc_ref, wl_ref, bl_ref,
                out_ref, hn_ref, cn_ref):
    dot = _DOT
    ndis = -dis_ref[...]
    g = (g0_ref[...] + dot(ndis * tx_ref[...], w1_ref[...])
         + dot(ndis * th_ref[...], w3_ref[...]))
    c_old = c_ref[...]
    wc = wc_ref[...]
    gi = jax.nn.sigmoid(g[:, 0:F] + wc[0:1, :] * c_old)
    gf = jax.nn.sigmoid(g[:, F:2 * F] + wc[1:2, :] * c_old)
    gt = jnp.tanh(g[:, 2 * F:3 * F])
    c_new = gf * c_old + gi * gt
    go = jax.nn.sigmoid(g[:, 3 * F:4 * F] + wc[2:3, :] * c_new)
    h_new = go * jnp.tanh(c_new)
    cn_ref[...] = c_new
    hn_ref[...] = h_new
    out_ref[...] = dot(h_new, wl_ref[...]) + bl_ref[...]


def _dense(g0, tx1x, tx1h, c, dis, w1, w3, wc, wl, bl):
    nblk = N // _RB
    row_spec = pl.BlockSpec((_RB, F), lambda i: (i, 0))
    full = lambda shape: pl.BlockSpec(shape, lambda i: (0,) * len(shape))
    return pl.pallas_call(
        _dense_body,
        grid=(nblk,),
        in_specs=[
            pl.BlockSpec((_RB, 4 * F), lambda i: (i, 0)),
            row_spec, row_spec, row_spec,
            pl.BlockSpec((_RB, 1), lambda i: (i, 0)),
            full((F, 4 * F)), full((F, 4 * F)),
            full((3, F)), full((F, 1)), full((1, 1)),
        ],
        out_specs=[
            pl.BlockSpec((_RB, 1), lambda i: (i, 0)),
            row_spec, row_spec,
        ],
        out_shape=[
            jax.ShapeDtypeStruct((N, 1), jnp.float32),
            jax.ShapeDtypeStruct((N, F), jnp.float32),
            jax.ShapeDtypeStruct((N, F), jnp.float32),
        ],
    )(g0, tx1x, tx1h, c, dis, w1, w3, wc, wl, bl)


def kernel(x, edge_index, edge_weight, h, c, Wx0, Wx1, bx, Wh0, Wh1, bh,
           wc, bg, W_lin, b_lin):
    src = edge_index[0]
    dst = edge_index[1]

    w0 = jnp.transpose(Wx0, (1, 0, 2)).reshape(F, 4 * F)
    w1 = jnp.transpose(Wx1, (1, 0, 2)).reshape(F, 4 * F)
    w2 = jnp.transpose(Wh0, (1, 0, 2)).reshape(F, 4 * F)
    w3 = jnp.transpose(Wh1, (1, 0, 2)).reshape(F, 4 * F)
    bias = (bx + bh + bg).reshape(1, 4 * F)
    bl = b_lin.reshape(1, 1)

    degp = _deg_partials(src, edge_weight).reshape(32, N)
    dis = _compute_dis(degp).reshape(N, 1)
    xs, hs = _prescale(dis, x, h)
    dst2 = dst.reshape(_NROW, _CH)
    tx1x, tx1h = _edge_pass(xs, hs, src, dst2, edge_weight)
    g0 = _dense0(x, h, w0, w2, bias)

    out, h_new, c_new = _dense(g0, tx1x, tx1h, c, dis, w1, w3, wc, W_lin, bl)
    return (out, h_new, c_new)
